# baseline (speedup 1.0000x reference)
import functools

import jax
import jax.numpy as jnp
from jax import lax
from jax.experimental import pallas as pl
from jax.experimental.pallas import tpu as pltpu
from jax.experimental.pallas import tpu_sc as plsc

NC = 2
NS = 16
NW = NC * NS
LANES = 16
CH = 128


def _sc_degree(dst_r, ones16, zeros16, n_pad):
  K = dst_r.shape[1]
  rpt = n_pad // NS
  mesh = plsc.VectorSubcoreMesh(core_axis_name="c", subcore_axis_name="s",
                                num_cores=NC)

  @functools.partial(
      pl.kernel,
      out_type=jax.ShapeDtypeStruct((NC, n_pad, LANES), jnp.float32),
      mesh=mesh,
      scratch_types=[
          pltpu.VMEM((K, CH), jnp.int32),
          pltpu.VMEM((CH, LANES), jnp.float32),
          pltpu.VMEM_SHARED((n_pad, LANES), jnp.float32),
      ],
      compiler_params=pltpu.CompilerParams(use_tc_tiling_on_sc=False),
  )
  def deg_kernel(dst_hbm, ones_hbm, zeros_hbm, out_hbm, idx_v, ones_v, acc_sh):
    c = lax.axis_index("c")
    s = lax.axis_index("s")
    wid = c * NS + s
    pltpu.sync_copy(dst_hbm.at[wid], idx_v)
    pltpu.sync_copy(ones_hbm, ones_v)
    rows = pl.ds(s * rpt, rpt)
    pltpu.sync_copy(zeros_hbm.at[rows], acc_sh.at[rows])
    plsc.subcore_barrier()

    def body(j, carry):
      pltpu.sync_copy(ones_v, acc_sh.at[idx_v.at[j]], add=True)
      return carry

    lax.fori_loop(0, K, body, 0)
    plsc.subcore_barrier()
    pltpu.sync_copy(acc_sh.at[rows], out_hbm.at[c, rows])

  return deg_kernel(dst_r, ones16, zeros16)


def _sc_propagate(table, src_r, dst_r, zerosC, n_pad, C):
  K = src_r.shape[1]
  rpt = n_pad // NS
  mesh = plsc.VectorSubcoreMesh(core_axis_name="c", subcore_axis_name="s",
                                num_cores=NC)

  @functools.partial(
      pl.kernel,
      out_type=jax.ShapeDtypeStruct((NC, n_pad, C), jnp.float32),
      mesh=mesh,
      scratch_types=[
          pltpu.VMEM((K, CH), jnp.int32),
          pltpu.VMEM((K, CH), jnp.int32),
          pltpu.VMEM((CH, C), jnp.float32),
          pltpu.VMEM((CH, C), jnp.float32),
          pltpu.VMEM_SHARED((n_pad, C), jnp.float32),
          pltpu.SemaphoreType.DMA,
      ],
      compiler_params=pltpu.CompilerParams(use_tc_tiling_on_sc=False),
  )
  def prop_kernel(table_hbm, src_hbm, dst_hbm, zeros_hbm, out_hbm,
                  isrc_v, idst_v, rows0_v, rows1_v, acc_sh, sem):
    c = lax.axis_index("c")
    s = lax.axis_index("s")
    wid = c * NS + s
    pltpu.sync_copy(src_hbm.at[wid], isrc_v)
    pltpu.sync_copy(dst_hbm.at[wid], idst_v)
    rows = pl.ds(s * rpt, rpt)
    pltpu.sync_copy(zeros_hbm.at[rows], acc_sh.at[rows])
    plsc.subcore_barrier()

    pltpu.async_copy(table_hbm.at[isrc_v.at[0]], rows0_v, sem)

    def body(g, carry):
      j0 = 2 * g
      j1 = j0 + 1
      pltpu.make_async_copy(table_hbm.at[isrc_v.at[j0]], rows0_v, sem).wait()
      pltpu.async_copy(table_hbm.at[isrc_v.at[j1]], rows1_v, sem)
      pltpu.sync_copy(rows0_v, acc_sh.at[idst_v.at[j0]], add=True)
      pltpu.make_async_copy(table_hbm.at[isrc_v.at[j1]], rows1_v, sem).wait()

      @pl.when(j1 + 1 < K)
      def _():
        pltpu.async_copy(table_hbm.at[isrc_v.at[j1 + 1]], rows0_v, sem)

      pltpu.sync_copy(rows1_v, acc_sh.at[idst_v.at[j1]], add=True)
      return carry

    lax.fori_loop(0, K // 2, body, 0)
    plsc.subcore_barrier()
    pltpu.sync_copy(acc_sh.at[rows], out_hbm.at[c, rows])

  return prop_kernel(table, src_r, dst_r, zerosC)


def _tc_matmul(x_pad, W):
  n_pad, D = x_pad.shape
  C = W.shape[0]
  nblk = 8
  rb = n_pad // nblk

  def mm_kernel(x_ref, w_ref, o_ref):
    o_ref[...] = lax.dot_general(
        x_ref[...], w_ref[...], (((1,), (1,)), ((), ())),
        preferred_element_type=jnp.float32)

  return pl.pallas_call(
      mm_kernel,
      grid=(nblk,),
      in_specs=[
          pl.BlockSpec((rb, D), lambda i: (i, 0)),
          pl.BlockSpec((C, D), lambda i: (0, 0)),
      ],
      out_specs=pl.BlockSpec((rb, C), lambda i: (i, 0)),
      out_shape=jax.ShapeDtypeStruct((n_pad, C), jnp.float32),
  )(x_pad, W)


def _tc_scale1(deg_parts, y):
  _, n_pad, _ = deg_parts.shape
  C = y.shape[1]
  nblk = 8
  rb = n_pad // nblk

  def k(p_ref, y_ref, dinv_ref, g1_ref):
    d = p_ref[0] + p_ref[1] + 1.0
    dinv = lax.rsqrt(d)
    dinv_ref[...] = dinv
    g1_ref[...] = dinv[:, :1] * y_ref[...]

  return pl.pallas_call(
      k,
      grid=(nblk,),
      in_specs=[
          pl.BlockSpec((NC, rb, LANES), lambda i: (0, i, 0)),
          pl.BlockSpec((rb, C), lambda i: (i, 0)),
      ],
      out_specs=[
          pl.BlockSpec((rb, LANES), lambda i: (i, 0)),
          pl.BlockSpec((rb, C), lambda i: (i, 0)),
      ],
      out_shape=[
          jax.ShapeDtypeStruct((n_pad, LANES), jnp.float32),
          jax.ShapeDtypeStruct((n_pad, C), jnp.float32),
      ],
  )(deg_parts, y)


def _tc_scale2(s1_parts, g1, dinv16):
  _, n_pad, C = s1_parts.shape
  nblk = 8
  rb = n_pad // nblk

  def k(sp_ref, g1_ref, dinv_ref, g2_ref):
    s = sp_ref[0] + sp_ref[1]
    di = dinv_ref[...][:, :1]
    g2_ref[...] = (di * di) * (s + g1_ref[...])

  return pl.pallas_call(
      k,
      grid=(nblk,),
      in_specs=[
          pl.BlockSpec((NC, rb, C), lambda i: (0, i, 0)),
          pl.BlockSpec((rb, C), lambda i: (i, 0)),
          pl.BlockSpec((rb, LANES), lambda i: (i, 0)),
      ],
      out_specs=pl.BlockSpec((rb, C), lambda i: (i, 0)),
      out_shape=jax.ShapeDtypeStruct((n_pad, C), jnp.float32),
  )(s1_parts, g1, dinv16)


def _tc_final(s2_parts, g2, dinv16, b2d):
  _, n_pad, C = s2_parts.shape
  nblk = 8
  rb = n_pad // nblk

  def k(sp_ref, g2_ref, dinv_ref, b_ref, o_ref):
    s = sp_ref[0] + sp_ref[1]
    di = dinv_ref[...][:, :1]
    o_ref[...] = di * (s + g2_ref[...]) + b_ref[...]

  return pl.pallas_call(
      k,
      grid=(nblk,),
      in_specs=[
          pl.BlockSpec((NC, rb, C), lambda i: (0, i, 0)),
          pl.BlockSpec((rb, C), lambda i: (i, 0)),
          pl.BlockSpec((rb, LANES), lambda i: (i, 0)),
          pl.BlockSpec((1, C), lambda i: (0, 0)),
      ],
      out_specs=pl.BlockSpec((rb, C), lambda i: (i, 0)),
      out_shape=jax.ShapeDtypeStruct((n_pad, C), jnp.float32),
  )(s2_parts, g2, dinv16, b2d)


def kernel(x, edge_index, W, b):
  N, D = x.shape
  C = W.shape[0]
  E = edge_index.shape[1]

  K = -(-E // (NW * CH))
  K = K + (K % 2)
  e_pad = NW * K * CH
  n_pad = -(-(N + 1) // (NS * 8)) * (NS * 8)

  ei = edge_index.astype(jnp.int32)
  if e_pad > E:
    pad = jnp.full((2, e_pad - E), N, dtype=jnp.int32)
    ei = jnp.concatenate([ei, pad], axis=1)
  src_r = ei[0].reshape(NW, K, CH)
  dst_r = ei[1].reshape(NW, K, CH)

  x_pad = jnp.zeros((n_pad, D), jnp.float32).at[:N].set(x)
  ones16 = jnp.ones((CH, LANES), jnp.float32)
  zeros16 = jnp.zeros((n_pad, LANES), jnp.float32)
  zerosC = jnp.zeros((n_pad, C), jnp.float32)

  deg_parts = _sc_degree(dst_r, ones16, zeros16, n_pad)
  y = _tc_matmul(x_pad, W)
  dinv16, g1 = _tc_scale1(deg_parts, y)
  s1 = _sc_propagate(g1, src_r, dst_r, zerosC, n_pad, C)
  g2 = _tc_scale2(s1, g1, dinv16)
  s2 = _sc_propagate(g2, src_r, dst_r, zerosC, n_pad, C)
  out_pad = _tc_final(s2, g2, dinv16, b.reshape(1, C))
  return out_pad[:N]


# speedup vs baseline: 1.1262x; 1.1262x over previous
import functools

import jax
import jax.numpy as jnp
from jax import lax
from jax.experimental import pallas as pl
from jax.experimental.pallas import tpu as pltpu
from jax.experimental.pallas import tpu_sc as plsc

NC = 2
NS = 16
NW = NC * NS
LANES = 16
CH = 128
NBUF = 4


def _sc_degree(dst_r, ones16, zeros16, n_pad):
  K = dst_r.shape[1]
  rpt = n_pad // NS
  mesh = plsc.VectorSubcoreMesh(core_axis_name="c", subcore_axis_name="s",
                                num_cores=NC)

  @functools.partial(
      pl.kernel,
      out_type=jax.ShapeDtypeStruct((NC, n_pad, LANES), jnp.float32),
      mesh=mesh,
      scratch_types=[
          pltpu.VMEM((K, CH), jnp.int32),
          pltpu.VMEM((CH, LANES), jnp.float32),
          pltpu.VMEM_SHARED((n_pad, LANES), jnp.float32),
      ],
      compiler_params=pltpu.CompilerParams(use_tc_tiling_on_sc=False),
  )
  def deg_kernel(dst_hbm, ones_hbm, zeros_hbm, out_hbm, idx_v, ones_v, acc_sh):
    c = lax.axis_index("c")
    s = lax.axis_index("s")
    wid = c * NS + s
    pltpu.sync_copy(dst_hbm.at[wid], idx_v)
    pltpu.sync_copy(ones_hbm, ones_v)
    rows = pl.ds(s * rpt, rpt)
    pltpu.sync_copy(zeros_hbm.at[rows], acc_sh.at[rows])
    plsc.subcore_barrier()

    def body(j, carry):
      pltpu.sync_copy(ones_v, acc_sh.at[idx_v.at[j]], add=True)
      return carry

    lax.fori_loop(0, K, body, 0)
    plsc.subcore_barrier()
    pltpu.sync_copy(acc_sh.at[rows], out_hbm.at[c, rows])

  return deg_kernel(dst_r, ones16, zeros16)


def _sc_propagate(table, src_r, dst_r, zerosC, n_pad, C):
  K = src_r.shape[1]
  rpt = n_pad // NS
  mesh = plsc.VectorSubcoreMesh(core_axis_name="c", subcore_axis_name="s",
                                num_cores=NC)

  @functools.partial(
      pl.kernel,
      out_type=jax.ShapeDtypeStruct((NC, n_pad, C), jnp.float32),
      mesh=mesh,
      scratch_types=[
          pltpu.VMEM((K, CH), jnp.int32),
          pltpu.VMEM((K, CH), jnp.int32),
          [pltpu.VMEM((CH, C), jnp.float32) for _ in range(NBUF)],
          pltpu.VMEM_SHARED((n_pad, C), jnp.float32),
          pltpu.SemaphoreType.DMA,
      ],
      compiler_params=pltpu.CompilerParams(use_tc_tiling_on_sc=False),
  )
  def prop_kernel(table_hbm, src_hbm, dst_hbm, zeros_hbm, out_hbm,
                  isrc_v, idst_v, rows_bufs, acc_sh, sem):
    c = lax.axis_index("c")
    s = lax.axis_index("s")
    wid = c * NS + s
    pltpu.sync_copy(src_hbm.at[wid], isrc_v)
    pltpu.sync_copy(dst_hbm.at[wid], idst_v)
    rows = pl.ds(s * rpt, rpt)
    pltpu.sync_copy(zeros_hbm.at[rows], acc_sh.at[rows])
    plsc.subcore_barrier()

    def body(g, carry):
      j0 = NBUF * g
      for b in range(NBUF):
        pltpu.async_copy(table_hbm.at[isrc_v.at[j0 + b]], rows_bufs[b], sem)
      for b in range(NBUF):
        pltpu.make_async_copy(
            table_hbm.at[isrc_v.at[j0 + b]], rows_bufs[b], sem).wait()
        pltpu.sync_copy(rows_bufs[b], acc_sh.at[idst_v.at[j0 + b]], add=True)
      return carry

    lax.fori_loop(0, K // NBUF, body, 0)
    plsc.subcore_barrier()
    pltpu.sync_copy(acc_sh.at[rows], out_hbm.at[c, rows])

  return prop_kernel(table, src_r, dst_r, zerosC)


def _tc_matmul(x_pad, W):
  n_pad, D = x_pad.shape
  C = W.shape[0]
  nblk = 8
  rb = n_pad // nblk

  def mm_kernel(x_ref, w_ref, o_ref):
    o_ref[...] = lax.dot_general(
        x_ref[...], w_ref[...], (((1,), (1,)), ((), ())),
        preferred_element_type=jnp.float32)

  return pl.pallas_call(
      mm_kernel,
      grid=(nblk,),
      in_specs=[
          pl.BlockSpec((rb, D), lambda i: (i, 0)),
          pl.BlockSpec((C, D), lambda i: (0, 0)),
      ],
      out_specs=pl.BlockSpec((rb, C), lambda i: (i, 0)),
      out_shape=jax.ShapeDtypeStruct((n_pad, C), jnp.float32),
  )(x_pad, W)


def _tc_scale1(deg_parts, y):
  _, n_pad, _ = deg_parts.shape
  C = y.shape[1]
  nblk = 8
  rb = n_pad // nblk

  def k(p_ref, y_ref, dinv_ref, g1_ref):
    d = p_ref[0] + p_ref[1] + 1.0
    dinv = lax.rsqrt(d)
    dinv_ref[...] = dinv
    g1_ref[...] = dinv[:, :1] * y_ref[...]

  return pl.pallas_call(
      k,
      grid=(nblk,),
      in_specs=[
          pl.BlockSpec((NC, rb, LANES), lambda i: (0, i, 0)),
          pl.BlockSpec((rb, C), lambda i: (i, 0)),
      ],
      out_specs=[
          pl.BlockSpec((rb, LANES), lambda i: (i, 0)),
          pl.BlockSpec((rb, C), lambda i: (i, 0)),
      ],
      out_shape=[
          jax.ShapeDtypeStruct((n_pad, LANES), jnp.float32),
          jax.ShapeDtypeStruct((n_pad, C), jnp.float32),
      ],
  )(deg_parts, y)


def _tc_scale2(s1_parts, g1, dinv16):
  _, n_pad, C = s1_parts.shape
  nblk = 8
  rb = n_pad // nblk

  def k(sp_ref, g1_ref, dinv_ref, g2_ref):
    s = sp_ref[0] + sp_ref[1]
    di = dinv_ref[...][:, :1]
    g2_ref[...] = (di * di) * (s + g1_ref[...])

  return pl.pallas_call(
      k,
      grid=(nblk,),
      in_specs=[
          pl.BlockSpec((NC, rb, C), lambda i: (0, i, 0)),
          pl.BlockSpec((rb, C), lambda i: (i, 0)),
          pl.BlockSpec((rb, LANES), lambda i: (i, 0)),
      ],
      out_specs=pl.BlockSpec((rb, C), lambda i: (i, 0)),
      out_shape=jax.ShapeDtypeStruct((n_pad, C), jnp.float32),
  )(s1_parts, g1, dinv16)


def _tc_final(s2_parts, g2, dinv16, b2d):
  _, n_pad, C = s2_parts.shape
  nblk = 8
  rb = n_pad // nblk

  def k(sp_ref, g2_ref, dinv_ref, b_ref, o_ref):
    s = sp_ref[0] + sp_ref[1]
    di = dinv_ref[...][:, :1]
    o_ref[...] = di * (s + g2_ref[...]) + b_ref[...]

  return pl.pallas_call(
      k,
      grid=(nblk,),
      in_specs=[
          pl.BlockSpec((NC, rb, C), lambda i: (0, i, 0)),
          pl.BlockSpec((rb, C), lambda i: (i, 0)),
          pl.BlockSpec((rb, LANES), lambda i: (i, 0)),
          pl.BlockSpec((1, C), lambda i: (0, 0)),
      ],
      out_specs=pl.BlockSpec((rb, C), lambda i: (i, 0)),
      out_shape=jax.ShapeDtypeStruct((n_pad, C), jnp.float32),
  )(s2_parts, g2, dinv16, b2d)


def kernel(x, edge_index, W, b):
  N, D = x.shape
  C = W.shape[0]
  E = edge_index.shape[1]

  K = -(-E // (NW * CH))
  K = -(-K // NBUF) * NBUF
  e_pad = NW * K * CH
  n_pad = -(-(N + 1) // (NS * 8)) * (NS * 8)

  ei = edge_index.astype(jnp.int32)
  if e_pad > E:
    pad = jnp.full((2, e_pad - E), N, dtype=jnp.int32)
    ei = jnp.concatenate([ei, pad], axis=1)
  src_r = ei[0].reshape(NW, K, CH)
  dst_r = ei[1].reshape(NW, K, CH)

  x_pad = jnp.zeros((n_pad, D), jnp.float32).at[:N].set(x)
  ones16 = jnp.ones((CH, LANES), jnp.float32)
  zeros16 = jnp.zeros((n_pad, LANES), jnp.float32)
  zerosC = jnp.zeros((n_pad, C), jnp.float32)

  deg_parts = _sc_degree(dst_r, ones16, zeros16, n_pad)
  y = _tc_matmul(x_pad, W)
  dinv16, g1 = _tc_scale1(deg_parts, y)
  s1 = _sc_propagate(g1, src_r, dst_r, zerosC, n_pad, C)
  g2 = _tc_scale2(s1, g1, dinv16)
  s2 = _sc_propagate(g2, src_r, dst_r, zerosC, n_pad, C)
  out_pad = _tc_final(s2, g2, dinv16, b.reshape(1, C))
  return out_pad[:N]


# speedup vs baseline: 1.8670x; 1.6578x over previous
import functools

import jax
import jax.numpy as jnp
from jax import lax
from jax.experimental import pallas as pl
from jax.experimental.pallas import tpu as pltpu
from jax.experimental.pallas import tpu_sc as plsc

NC = 2
NS = 16
NW = NC * NS
LANES = 16
CH = 128
NBUF = 4


def _sc_degree(dst_r, ones16, zeros16, n_pad):
  K = dst_r.shape[1]
  rpt = n_pad // NS
  mesh = plsc.VectorSubcoreMesh(core_axis_name="c", subcore_axis_name="s",
                                num_cores=NC)

  @functools.partial(
      pl.kernel,
      out_type=jax.ShapeDtypeStruct((NC, n_pad, LANES), jnp.float32),
      mesh=mesh,
      scratch_types=[
          pltpu.VMEM((K, CH), jnp.int32),
          pltpu.VMEM((CH, LANES), jnp.float32),
          pltpu.VMEM_SHARED((n_pad, LANES), jnp.float32),
      ],
      compiler_params=pltpu.CompilerParams(use_tc_tiling_on_sc=False),
  )
  def deg_kernel(dst_hbm, ones_hbm, zeros_hbm, out_hbm, idx_v, ones_v, acc_sh):
    c = lax.axis_index("c")
    s = lax.axis_index("s")
    wid = c * NS + s
    pltpu.sync_copy(dst_hbm.at[wid], idx_v)
    pltpu.sync_copy(ones_hbm, ones_v)
    rows = pl.ds(s * rpt, rpt)
    pltpu.sync_copy(zeros_hbm.at[rows], acc_sh.at[rows])
    plsc.subcore_barrier()

    def body(j, carry):
      pltpu.sync_copy(ones_v, acc_sh.at[idx_v.at[j]], add=True)
      return carry

    lax.fori_loop(0, K, body, 0)
    plsc.subcore_barrier()
    pltpu.sync_copy(acc_sh.at[rows], out_hbm.at[c, rows])

  return deg_kernel(dst_r, ones16, zeros16)


def _sc_propagate(table, src_r, dst_r, zerosC, n_pad, C):
  K = src_r.shape[1]
  rpt = n_pad // NS
  mesh = plsc.VectorSubcoreMesh(core_axis_name="c", subcore_axis_name="s",
                                num_cores=NC)

  @functools.partial(
      pl.kernel,
      out_type=jax.ShapeDtypeStruct((NC, n_pad, C), jnp.float32),
      mesh=mesh,
      scratch_types=[
          pltpu.VMEM((K, CH), jnp.int32),
          pltpu.VMEM((K, CH), jnp.int32),
          pltpu.VMEM((CH, C), jnp.float32),
          pltpu.VMEM_SHARED((n_pad, C), jnp.float32),
          pltpu.VMEM_SHARED((n_pad, C), jnp.float32),
          pltpu.SemaphoreType.DMA,
      ],
      compiler_params=pltpu.CompilerParams(use_tc_tiling_on_sc=False),
  )
  def prop_kernel(table_hbm, src_hbm, dst_hbm, zeros_hbm, out_hbm,
                  isrc_v, idst_v, rows_v, table_sh, acc_sh, sem):
    c = lax.axis_index("c")
    s = lax.axis_index("s")
    wid = c * NS + s
    pltpu.sync_copy(src_hbm.at[wid], isrc_v)
    pltpu.sync_copy(dst_hbm.at[wid], idst_v)
    rows = pl.ds(s * rpt, rpt)
    pltpu.sync_copy(table_hbm.at[rows], table_sh.at[rows])
    pltpu.sync_copy(zeros_hbm.at[rows], acc_sh.at[rows])
    plsc.subcore_barrier()

    def body(j, carry):
      pltpu.async_copy(table_sh.at[isrc_v.at[j]], rows_v, sem).wait()
      pltpu.sync_copy(rows_v, acc_sh.at[idst_v.at[j]], add=True)
      return carry

    lax.fori_loop(0, K, body, 0)
    plsc.subcore_barrier()
    pltpu.sync_copy(acc_sh.at[rows], out_hbm.at[c, rows])

  return prop_kernel(table, src_r, dst_r, zerosC)


def _tc_matmul(x_pad, W):
  n_pad, D = x_pad.shape
  C = W.shape[0]
  nblk = 8
  rb = n_pad // nblk

  def mm_kernel(x_ref, w_ref, o_ref):
    o_ref[...] = lax.dot_general(
        x_ref[...], w_ref[...], (((1,), (1,)), ((), ())),
        preferred_element_type=jnp.float32)

  return pl.pallas_call(
      mm_kernel,
      grid=(nblk,),
      in_specs=[
          pl.BlockSpec((rb, D), lambda i: (i, 0)),
          pl.BlockSpec((C, D), lambda i: (0, 0)),
      ],
      out_specs=pl.BlockSpec((rb, C), lambda i: (i, 0)),
      out_shape=jax.ShapeDtypeStruct((n_pad, C), jnp.float32),
  )(x_pad, W)


def _tc_scale1(deg_parts, y):
  _, n_pad, _ = deg_parts.shape
  C = y.shape[1]
  nblk = 8
  rb = n_pad // nblk

  def k(p_ref, y_ref, dinv_ref, g1_ref):
    d = p_ref[0] + p_ref[1] + 1.0
    dinv = lax.rsqrt(d)
    dinv_ref[...] = dinv
    g1_ref[...] = dinv[:, :1] * y_ref[...]

  return pl.pallas_call(
      k,
      grid=(nblk,),
      in_specs=[
          pl.BlockSpec((NC, rb, LANES), lambda i: (0, i, 0)),
          pl.BlockSpec((rb, C), lambda i: (i, 0)),
      ],
      out_specs=[
          pl.BlockSpec((rb, LANES), lambda i: (i, 0)),
          pl.BlockSpec((rb, C), lambda i: (i, 0)),
      ],
      out_shape=[
          jax.ShapeDtypeStruct((n_pad, LANES), jnp.float32),
          jax.ShapeDtypeStruct((n_pad, C), jnp.float32),
      ],
  )(deg_parts, y)


def _tc_scale2(s1_parts, g1, dinv16):
  _, n_pad, C = s1_parts.shape
  nblk = 8
  rb = n_pad // nblk

  def k(sp_ref, g1_ref, dinv_ref, g2_ref):
    s = sp_ref[0] + sp_ref[1]
    di = dinv_ref[...][:, :1]
    g2_ref[...] = (di * di) * (s + g1_ref[...])

  return pl.pallas_call(
      k,
      grid=(nblk,),
      in_specs=[
          pl.BlockSpec((NC, rb, C), lambda i: (0, i, 0)),
          pl.BlockSpec((rb, C), lambda i: (i, 0)),
          pl.BlockSpec((rb, LANES), lambda i: (i, 0)),
      ],
      out_specs=pl.BlockSpec((rb, C), lambda i: (i, 0)),
      out_shape=jax.ShapeDtypeStruct((n_pad, C), jnp.float32),
  )(s1_parts, g1, dinv16)


def _tc_final(s2_parts, g2, dinv16, b2d):
  _, n_pad, C = s2_parts.shape
  nblk = 8
  rb = n_pad // nblk

  def k(sp_ref, g2_ref, dinv_ref, b_ref, o_ref):
    s = sp_ref[0] + sp_ref[1]
    di = dinv_ref[...][:, :1]
    o_ref[...] = di * (s + g2_ref[...]) + b_ref[...]

  return pl.pallas_call(
      k,
      grid=(nblk,),
      in_specs=[
          pl.BlockSpec((NC, rb, C), lambda i: (0, i, 0)),
          pl.BlockSpec((rb, C), lambda i: (i, 0)),
          pl.BlockSpec((rb, LANES), lambda i: (i, 0)),
          pl.BlockSpec((1, C), lambda i: (0, 0)),
      ],
      out_specs=pl.BlockSpec((rb, C), lambda i: (i, 0)),
      out_shape=jax.ShapeDtypeStruct((n_pad, C), jnp.float32),
  )(s2_parts, g2, dinv16, b2d)


def kernel(x, edge_index, W, b):
  N, D = x.shape
  C = W.shape[0]
  E = edge_index.shape[1]

  K = -(-E // (NW * CH))
  K = -(-K // NBUF) * NBUF
  e_pad = NW * K * CH
  n_pad = -(-(N + 1) // (NS * 8)) * (NS * 8)

  ei = edge_index.astype(jnp.int32)
  if e_pad > E:
    pad = jnp.full((2, e_pad - E), N, dtype=jnp.int32)
    ei = jnp.concatenate([ei, pad], axis=1)
  src_r = ei[0].reshape(NW, K, CH)
  dst_r = ei[1].reshape(NW, K, CH)

  x_pad = jnp.zeros((n_pad, D), jnp.float32).at[:N].set(x)
  ones16 = jnp.ones((CH, LANES), jnp.float32)
  zeros16 = jnp.zeros((n_pad, LANES), jnp.float32)
  zerosC = jnp.zeros((n_pad, C), jnp.float32)

  deg_parts = _sc_degree(dst_r, ones16, zeros16, n_pad)
  y = _tc_matmul(x_pad, W)
  dinv16, g1 = _tc_scale1(deg_parts, y)
  s1 = _sc_propagate(g1, src_r, dst_r, zerosC, n_pad, C)
  g2 = _tc_scale2(s1, g1, dinv16)
  s2 = _sc_propagate(g2, src_r, dst_r, zerosC, n_pad, C)
  out_pad = _tc_final(s2, g2, dinv16, b.reshape(1, C))
  return out_pad[:N]


# speedup vs baseline: 1.8739x; 1.0037x over previous
import functools

import jax
import jax.numpy as jnp
from jax import lax
from jax.experimental import pallas as pl
from jax.experimental.pallas import tpu as pltpu
from jax.experimental.pallas import tpu_sc as plsc

NC = 2
NS = 16
NW = NC * NS
LANES = 16
CH = 128
NBUF = 4


def _sc_degree(dst_r, ones16, zeros16, n_pad):
  K = dst_r.shape[1]
  rpt = n_pad // NS
  mesh = plsc.VectorSubcoreMesh(core_axis_name="c", subcore_axis_name="s",
                                num_cores=NC)

  @functools.partial(
      pl.kernel,
      out_type=jax.ShapeDtypeStruct((NC, n_pad, LANES), jnp.float32),
      mesh=mesh,
      scratch_types=[
          pltpu.VMEM((K, CH), jnp.int32),
          pltpu.VMEM((CH, LANES), jnp.float32),
          pltpu.VMEM_SHARED((n_pad, LANES), jnp.float32),
      ],
      compiler_params=pltpu.CompilerParams(use_tc_tiling_on_sc=False),
  )
  def deg_kernel(dst_hbm, ones_hbm, zeros_hbm, out_hbm, idx_v, ones_v, acc_sh):
    c = lax.axis_index("c")
    s = lax.axis_index("s")
    wid = c * NS + s
    pltpu.sync_copy(dst_hbm.at[wid], idx_v)
    pltpu.sync_copy(ones_hbm, ones_v)
    rows = pl.ds(s * rpt, rpt)
    pltpu.sync_copy(zeros_hbm.at[rows], acc_sh.at[rows])
    plsc.subcore_barrier()

    def body(j, carry):
      pltpu.sync_copy(ones_v, acc_sh.at[idx_v.at[j]], add=True)
      return carry

    lax.fori_loop(0, K, body, 0)
    plsc.subcore_barrier()
    pltpu.sync_copy(acc_sh.at[rows], out_hbm.at[c, rows])

  return deg_kernel(dst_r, ones16, zeros16)


def _sc_propagate(table, src_r, dst_r, zerosC, n_pad, C):
  K = src_r.shape[1]
  rpt = n_pad // NS
  mesh = plsc.VectorSubcoreMesh(core_axis_name="c", subcore_axis_name="s",
                                num_cores=NC)

  @functools.partial(
      pl.kernel,
      out_type=jax.ShapeDtypeStruct((NC, n_pad, C), jnp.float32),
      mesh=mesh,
      scratch_types=[
          pltpu.VMEM((K, CH), jnp.int32),
          pltpu.VMEM((K, CH), jnp.int32),
          pltpu.VMEM((CH, C), jnp.float32),
          pltpu.VMEM_SHARED((n_pad, C), jnp.float32),
          pltpu.VMEM_SHARED((n_pad, C), jnp.float32),
          pltpu.SemaphoreType.DMA,
      ],
      compiler_params=pltpu.CompilerParams(use_tc_tiling_on_sc=False),
  )
  def prop_kernel(table_hbm, src_hbm, dst_hbm, zeros_hbm, out_hbm,
                  isrc_v, idst_v, rows_v, table_sh, acc_sh, sem):
    c = lax.axis_index("c")
    s = lax.axis_index("s")
    wid = c * NS + s
    pltpu.sync_copy(src_hbm.at[wid], isrc_v)
    pltpu.sync_copy(dst_hbm.at[wid], idst_v)
    rows = pl.ds(s * rpt, rpt)
    pltpu.sync_copy(table_hbm.at[rows], table_sh.at[rows])
    pltpu.sync_copy(zeros_hbm.at[rows], acc_sh.at[rows])
    plsc.subcore_barrier()

    def body(j, carry):
      pltpu.async_copy(table_sh.at[isrc_v.at[j]], rows_v, sem).wait()
      pltpu.sync_copy(rows_v, acc_sh.at[idst_v.at[j]], add=True)
      return carry

    lax.fori_loop(0, K, body, 0)
    plsc.subcore_barrier()
    pltpu.sync_copy(acc_sh.at[rows], out_hbm.at[c, rows])

  return prop_kernel(table, src_r, dst_r, zerosC)


def _tc_matmul_scale1(deg_parts, x_pad, W):
  n_pad, D = x_pad.shape
  C = W.shape[0]
  nblk = 8
  rb = n_pad // nblk

  def k(p_ref, x_ref, w_ref, dinv_ref, g1_ref):
    y = lax.dot_general(
        x_ref[...], w_ref[...], (((1,), (1,)), ((), ())),
        preferred_element_type=jnp.float32)
    d = p_ref[0] + p_ref[1] + 1.0
    dinv = lax.rsqrt(d)
    dinv_ref[...] = dinv
    g1_ref[...] = dinv[:, :1] * y

  return pl.pallas_call(
      k,
      grid=(nblk,),
      in_specs=[
          pl.BlockSpec((NC, rb, LANES), lambda i: (0, i, 0)),
          pl.BlockSpec((rb, D), lambda i: (i, 0)),
          pl.BlockSpec((C, D), lambda i: (0, 0)),
      ],
      out_specs=[
          pl.BlockSpec((rb, LANES), lambda i: (i, 0)),
          pl.BlockSpec((rb, C), lambda i: (i, 0)),
      ],
      out_shape=[
          jax.ShapeDtypeStruct((n_pad, LANES), jnp.float32),
          jax.ShapeDtypeStruct((n_pad, C), jnp.float32),
      ],
  )(deg_parts, x_pad, W)


def _tc_scale2(s1_parts, g1, dinv16):
  _, n_pad, C = s1_parts.shape
  nblk = 8
  rb = n_pad // nblk

  def k(sp_ref, g1_ref, dinv_ref, g2_ref):
    s = sp_ref[0] + sp_ref[1]
    di = dinv_ref[...][:, :1]
    g2_ref[...] = (di * di) * (s + g1_ref[...])

  return pl.pallas_call(
      k,
      grid=(nblk,),
      in_specs=[
          pl.BlockSpec((NC, rb, C), lambda i: (0, i, 0)),
          pl.BlockSpec((rb, C), lambda i: (i, 0)),
          pl.BlockSpec((rb, LANES), lambda i: (i, 0)),
      ],
      out_specs=pl.BlockSpec((rb, C), lambda i: (i, 0)),
      out_shape=jax.ShapeDtypeStruct((n_pad, C), jnp.float32),
  )(s1_parts, g1, dinv16)


def _tc_final(s2_parts, g2, dinv16, b2d):
  _, n_pad, C = s2_parts.shape
  nblk = 8
  rb = n_pad // nblk

  def k(sp_ref, g2_ref, dinv_ref, b_ref, o_ref):
    s = sp_ref[0] + sp_ref[1]
    di = dinv_ref[...][:, :1]
    o_ref[...] = di * (s + g2_ref[...]) + b_ref[...]

  return pl.pallas_call(
      k,
      grid=(nblk,),
      in_specs=[
          pl.BlockSpec((NC, rb, C), lambda i: (0, i, 0)),
          pl.BlockSpec((rb, C), lambda i: (i, 0)),
          pl.BlockSpec((rb, LANES), lambda i: (i, 0)),
          pl.BlockSpec((1, C), lambda i: (0, 0)),
      ],
      out_specs=pl.BlockSpec((rb, C), lambda i: (i, 0)),
      out_shape=jax.ShapeDtypeStruct((n_pad, C), jnp.float32),
  )(s2_parts, g2, dinv16, b2d)


def kernel(x, edge_index, W, b):
  N, D = x.shape
  C = W.shape[0]
  E = edge_index.shape[1]

  K = -(-E // (NW * CH))
  K = -(-K // NBUF) * NBUF
  e_pad = NW * K * CH
  n_pad = -(-(N + 1) // (NS * 8)) * (NS * 8)

  ei = edge_index.astype(jnp.int32)
  if e_pad > E:
    pad = jnp.full((2, e_pad - E), N, dtype=jnp.int32)
    ei = jnp.concatenate([ei, pad], axis=1)
  src_r = ei[0].reshape(NW, K, CH)
  dst_r = ei[1].reshape(NW, K, CH)

  x_pad = jnp.zeros((n_pad, D), jnp.float32).at[:N].set(x)
  ones16 = jnp.ones((CH, LANES), jnp.float32)
  zeros16 = jnp.zeros((n_pad, LANES), jnp.float32)
  zerosC = jnp.zeros((n_pad, C), jnp.float32)

  deg_parts = _sc_degree(dst_r, ones16, zeros16, n_pad)
  dinv16, g1 = _tc_matmul_scale1(deg_parts, x_pad, W)
  s1 = _sc_propagate(g1, src_r, dst_r, zerosC, n_pad, C)
  g2 = _tc_scale2(s1, g1, dinv16)
  s2 = _sc_propagate(g2, src_r, dst_r, zerosC, n_pad, C)
  out_pad = _tc_final(s2, g2, dinv16, b.reshape(1, C))
  return out_pad[:N]


# speedup vs baseline: 2.2810x; 1.2172x over previous
import functools

import jax
import jax.numpy as jnp
from jax import lax
from jax.experimental import pallas as pl
from jax.experimental.pallas import tpu as pltpu
from jax.experimental.pallas import tpu_sc as plsc

NC = 2
NS = 16
NW = NC * NS
LANES = 16
CH = 128
NBUF = 4


def _sc_degree(dst_r, ones16, zeros16, n_pad):
  K = dst_r.shape[1]
  rpt = n_pad // NS
  mesh = plsc.VectorSubcoreMesh(core_axis_name="c", subcore_axis_name="s",
                                num_cores=NC)

  @functools.partial(
      pl.kernel,
      out_type=jax.ShapeDtypeStruct((NC, n_pad, LANES), jnp.float32),
      mesh=mesh,
      scratch_types=[
          pltpu.VMEM((K, CH), jnp.int32),
          pltpu.VMEM((CH, LANES), jnp.float32),
          pltpu.VMEM_SHARED((n_pad, LANES), jnp.float32),
      ],
      compiler_params=pltpu.CompilerParams(use_tc_tiling_on_sc=False),
  )
  def deg_kernel(dst_hbm, ones_hbm, zeros_hbm, out_hbm, idx_v, ones_v, acc_sh):
    c = lax.axis_index("c")
    s = lax.axis_index("s")
    wid = c * NS + s
    pltpu.sync_copy(dst_hbm.at[wid], idx_v)
    pltpu.sync_copy(ones_hbm, ones_v)
    rows = pl.ds(s * rpt, rpt)
    pltpu.sync_copy(zeros_hbm.at[rows], acc_sh.at[rows])
    plsc.subcore_barrier()

    def body(j, carry):
      pltpu.sync_copy(ones_v, acc_sh.at[idx_v.at[j]], add=True)
      return carry

    lax.fori_loop(0, K, body, 0)
    plsc.subcore_barrier()
    pltpu.sync_copy(acc_sh.at[rows], out_hbm.at[c, rows])

  return deg_kernel(dst_r, ones16, zeros16)


def _sc_propagate(table, src_r, dst_r, zerosC, n_pad, C):
  K = src_r.shape[1]
  rpt = n_pad // NS
  mesh = plsc.VectorSubcoreMesh(core_axis_name="c", subcore_axis_name="s",
                                num_cores=NC)

  @functools.partial(
      pl.kernel,
      out_type=jax.ShapeDtypeStruct((NC, n_pad, C), jnp.float32),
      mesh=mesh,
      scratch_types=[
          pltpu.VMEM((K, CH), jnp.int32),
          pltpu.VMEM((K, CH), jnp.int32),
          pltpu.VMEM((CH, C), jnp.float32),
          pltpu.VMEM((CH, C), jnp.float32),
          pltpu.VMEM_SHARED((n_pad, C), jnp.float32),
          pltpu.VMEM_SHARED((n_pad, C), jnp.float32),
          pltpu.SemaphoreType.DMA,
      ],
      compiler_params=pltpu.CompilerParams(use_tc_tiling_on_sc=False),
  )
  def prop_kernel(table_hbm, src_hbm, dst_hbm, zeros_hbm, out_hbm,
                  isrc_v, idst_v, rows_v, rows2_v, table_sh, acc_sh, sem):
    c = lax.axis_index("c")
    s = lax.axis_index("s")
    wid = c * NS + s
    pltpu.sync_copy(src_hbm.at[wid], isrc_v)
    pltpu.sync_copy(dst_hbm.at[wid], idst_v)
    rows = pl.ds(s * rpt, rpt)
    pltpu.sync_copy(table_hbm.at[rows], table_sh.at[rows])
    pltpu.sync_copy(zeros_hbm.at[rows], acc_sh.at[rows])
    plsc.subcore_barrier()

    pltpu.async_copy(table_sh.at[isrc_v.at[0]], rows_v, sem)

    def body(g, carry):
      j0 = 2 * g
      j1 = j0 + 1
      pltpu.make_async_copy(table_sh.at[isrc_v.at[j0]], rows_v, sem).wait()
      pltpu.async_copy(table_sh.at[isrc_v.at[j1]], rows2_v, sem)
      pltpu.sync_copy(rows_v, acc_sh.at[idst_v.at[j0]], add=True)
      pltpu.make_async_copy(table_sh.at[isrc_v.at[j1]], rows2_v, sem).wait()

      @pl.when(j1 + 1 < K)
      def _():
        pltpu.async_copy(table_sh.at[isrc_v.at[j1 + 1]], rows_v, sem)

      pltpu.sync_copy(rows2_v, acc_sh.at[idst_v.at[j1]], add=True)
      return carry

    lax.fori_loop(0, K // 2, body, 0)
    plsc.subcore_barrier()
    pltpu.sync_copy(acc_sh.at[rows], out_hbm.at[c, rows])

  return prop_kernel(table, src_r, dst_r, zerosC)


def _tc_matmul_scale1(deg_parts, x_pad, W):
  n_pad, D = x_pad.shape
  C = W.shape[0]
  nblk = 8
  rb = n_pad // nblk

  def k(p_ref, x_ref, w_ref, dinv_ref, g1_ref):
    y = lax.dot_general(
        x_ref[...], w_ref[...], (((1,), (1,)), ((), ())),
        preferred_element_type=jnp.float32)
    d = p_ref[0] + p_ref[1] + 1.0
    dinv = lax.rsqrt(d)
    dinv_ref[...] = dinv
    g1_ref[...] = dinv[:, :1] * y

  return pl.pallas_call(
      k,
      grid=(nblk,),
      in_specs=[
          pl.BlockSpec((NC, rb, LANES), lambda i: (0, i, 0)),
          pl.BlockSpec((rb, D), lambda i: (i, 0)),
          pl.BlockSpec((C, D), lambda i: (0, 0)),
      ],
      out_specs=[
          pl.BlockSpec((rb, LANES), lambda i: (i, 0)),
          pl.BlockSpec((rb, C), lambda i: (i, 0)),
      ],
      out_shape=[
          jax.ShapeDtypeStruct((n_pad, LANES), jnp.float32),
          jax.ShapeDtypeStruct((n_pad, C), jnp.float32),
      ],
  )(deg_parts, x_pad, W)


def _tc_scale2(s1_parts, g1, dinv16):
  _, n_pad, C = s1_parts.shape
  nblk = 8
  rb = n_pad // nblk

  def k(sp_ref, g1_ref, dinv_ref, g2_ref):
    s = sp_ref[0] + sp_ref[1]
    di = dinv_ref[...][:, :1]
    g2_ref[...] = (di * di) * (s + g1_ref[...])

  return pl.pallas_call(
      k,
      grid=(nblk,),
      in_specs=[
          pl.BlockSpec((NC, rb, C), lambda i: (0, i, 0)),
          pl.BlockSpec((rb, C), lambda i: (i, 0)),
          pl.BlockSpec((rb, LANES), lambda i: (i, 0)),
      ],
      out_specs=pl.BlockSpec((rb, C), lambda i: (i, 0)),
      out_shape=jax.ShapeDtypeStruct((n_pad, C), jnp.float32),
  )(s1_parts, g1, dinv16)


def _tc_final(s2_parts, g2, dinv16, b2d):
  _, n_pad, C = s2_parts.shape
  nblk = 8
  rb = n_pad // nblk

  def k(sp_ref, g2_ref, dinv_ref, b_ref, o_ref):
    s = sp_ref[0] + sp_ref[1]
    di = dinv_ref[...][:, :1]
    o_ref[...] = di * (s + g2_ref[...]) + b_ref[...]

  return pl.pallas_call(
      k,
      grid=(nblk,),
      in_specs=[
          pl.BlockSpec((NC, rb, C), lambda i: (0, i, 0)),
          pl.BlockSpec((rb, C), lambda i: (i, 0)),
          pl.BlockSpec((rb, LANES), lambda i: (i, 0)),
          pl.BlockSpec((1, C), lambda i: (0, 0)),
      ],
      out_specs=pl.BlockSpec((rb, C), lambda i: (i, 0)),
      out_shape=jax.ShapeDtypeStruct((n_pad, C), jnp.float32),
  )(s2_parts, g2, dinv16, b2d)


def kernel(x, edge_index, W, b):
  N, D = x.shape
  C = W.shape[0]
  E = edge_index.shape[1]

  K = -(-E // (NW * CH))
  K = -(-K // NBUF) * NBUF
  e_pad = NW * K * CH
  n_pad = -(-(N + 1) // (NS * 8)) * (NS * 8)

  ei = edge_index.astype(jnp.int32)
  if e_pad > E:
    pad = jnp.full((2, e_pad - E), N, dtype=jnp.int32)
    ei = jnp.concatenate([ei, pad], axis=1)
  src_r = ei[0].reshape(NW, K, CH)
  dst_r = ei[1].reshape(NW, K, CH)

  x_pad = jnp.zeros((n_pad, D), jnp.float32).at[:N].set(x)
  ones16 = jnp.ones((CH, LANES), jnp.float32)
  zeros16 = jnp.zeros((n_pad, LANES), jnp.float32)
  zerosC = jnp.zeros((n_pad, C), jnp.float32)

  deg_parts = _sc_degree(dst_r, ones16, zeros16, n_pad)
  dinv16, g1 = _tc_matmul_scale1(deg_parts, x_pad, W)
  s1 = _sc_propagate(g1, src_r, dst_r, zerosC, n_pad, C)
  g2 = _tc_scale2(s1, g1, dinv16)
  s2 = _sc_propagate(g2, src_r, dst_r, zerosC, n_pad, C)
  out_pad = _tc_final(s2, g2, dinv16, b.reshape(1, C))
  return out_pad[:N]
